# baseline (speedup 1.0000x reference)
import functools

import jax
import jax.numpy as jnp
from jax import lax
from jax.experimental import pallas as pl
from jax.experimental.pallas import tpu as pltpu
from jax.experimental.pallas import tpu_sc as plsc

VOCAB = 21
D = 128
VP = 32
B = 4096
L = 500
NLANES = 16
NW = 32
COLS_PER_W = B // NW
GROUPS = COLS_PER_W // NLANES

_mesh = plsc.VectorSubcoreMesh(core_axis_name="c", subcore_axis_name="s")


@functools.partial(
    pl.kernel,
    out_type=jax.ShapeDtypeStruct((NW, VP, COLS_PER_W), jnp.float32),
    mesh=_mesh,
    scratch_types=[
        pltpu.VMEM((L, COLS_PER_W), jnp.int32),
        pltpu.VMEM((VP, COLS_PER_W), jnp.float32),
    ],
    compiler_params=pltpu.CompilerParams(
        needs_layout_passes=False,
        use_tc_tiling_on_sc=False,
        skip_device_barrier=True,
    ),
)
def _sc_hist(idxt_hbm, cnt_hbm, idx_v, cnt_v):
    wid = lax.axis_index("s") * 2 + lax.axis_index("c")
    base = wid * COLS_PER_W
    pltpu.sync_copy(idxt_hbm.at[:, pl.ds(base, COLS_PER_W)], idx_v)

    zf = jnp.zeros((NLANES,), jnp.float32)

    @pl.loop(0, VP)
    def _zero(v):
        for j in range(COLS_PER_W // NLANES):
            cnt_v[v, pl.ds(j * NLANES, NLANES)] = zf

    iota16 = lax.iota(jnp.int32, NLANES)
    ones = jnp.ones((NLANES,), jnp.float32)

    for g in range(GROUPS):
        cvec = iota16 + (g * NLANES)

        @plsc.parallel_loop(0, L, unroll=8)
        def _acc(l, g=g, cvec=cvec):
            ids = idx_v[l, pl.ds(g * NLANES, NLANES)]
            plsc.addupdate_scatter(cnt_v, [ids, cvec], ones)

    pltpu.sync_copy(cnt_v, cnt_hbm.at[wid])


def _mm_body(cnt_ref, tab_ref, out_ref):
    out_ref[...] = lax.dot_general(
        cnt_ref[0], tab_ref[...],
        (((0,), (0,)), ((), ())),
        preferred_element_type=jnp.float32,
    ) * (1.0 / L)


def _tc_matmul(counts, tablep):
    return pl.pallas_call(
        _mm_body,
        grid=(NW,),
        in_specs=[
            pl.BlockSpec((1, VP, COLS_PER_W), lambda i: (i, 0, 0)),
            pl.BlockSpec((VP, D), lambda i: (0, 0)),
        ],
        out_specs=pl.BlockSpec((COLS_PER_W, D), lambda i: (i, 0)),
        out_shape=jax.ShapeDtypeStruct((B, D), jnp.float32),
    )(counts, tablep)


def kernel(indices, table):
    indices = indices.astype(jnp.int32)
    table = table.astype(jnp.float32)
    counts = _sc_hist(indices.T)
    tablep = jnp.concatenate(
        [table, jnp.zeros((VP - VOCAB, D), jnp.float32)], axis=0)
    return _tc_matmul(counts, tablep)


# speedup vs baseline: 1.1710x; 1.1710x over previous
import functools

import jax
import jax.numpy as jnp
from jax import lax
from jax.experimental import pallas as pl
from jax.experimental.pallas import tpu as pltpu
from jax.experimental.pallas import tpu_sc as plsc

VOCAB = 21
D = 128
VP = 32
B = 4096
L = 500
NLANES = 16
NW = 32
COLS_PER_W = B // NW
GROUPS = COLS_PER_W // NLANES

_mesh = plsc.VectorSubcoreMesh(core_axis_name="c", subcore_axis_name="s")


@functools.partial(
    pl.kernel,
    out_type=jax.ShapeDtypeStruct((B, VP), jnp.float32),
    mesh=_mesh,
    scratch_types=[
        pltpu.VMEM((L, COLS_PER_W), jnp.int32),
        pltpu.VMEM((COLS_PER_W, VP), jnp.float32),
    ],
    compiler_params=pltpu.CompilerParams(
        needs_layout_passes=False,
        use_tc_tiling_on_sc=False,
    ),
)
def _sc_hist(idxt_hbm, cnt_hbm, idx_v, cnt_v):
    wid = lax.axis_index("s") * 2 + lax.axis_index("c")
    base = wid * COLS_PER_W
    pltpu.sync_copy(idxt_hbm.at[:, pl.ds(base, COLS_PER_W)], idx_v)

    zf = jnp.zeros((NLANES,), jnp.float32)

    @pl.loop(0, COLS_PER_W)
    def _zero(c):
        cnt_v[c, pl.ds(0, NLANES)] = zf
        cnt_v[c, pl.ds(NLANES, NLANES)] = zf

    iota16 = lax.iota(jnp.int32, NLANES)
    ones = jnp.ones((NLANES,), jnp.float32)

    for g in range(GROUPS):
        cloc = iota16 + (g * NLANES)

        @plsc.parallel_loop(0, L, unroll=8)
        def _acc(l, g=g, cloc=cloc):
            ids = idx_v[l, pl.ds(g * NLANES, NLANES)]
            plsc.addupdate_scatter(cnt_v, [cloc, ids], ones)

    pltpu.sync_copy(cnt_v, cnt_hbm.at[pl.ds(base, COLS_PER_W)])


def _mm_body(cnt_ref, tab_ref, out_ref):
    out_ref[...] = lax.dot_general(
        cnt_ref[...], tab_ref[...],
        (((1,), (0,)), ((), ())),
        preferred_element_type=jnp.float32,
    ) * (1.0 / L)


_MM_BLK = 1024


def _tc_matmul(counts, tablep):
    return pl.pallas_call(
        _mm_body,
        grid=(B // _MM_BLK,),
        in_specs=[
            pl.BlockSpec((_MM_BLK, VP), lambda i: (i, 0)),
            pl.BlockSpec((VP, D), lambda i: (0, 0)),
        ],
        out_specs=pl.BlockSpec((_MM_BLK, D), lambda i: (i, 0)),
        out_shape=jax.ShapeDtypeStruct((B, D), jnp.float32),
    )(counts, tablep)


def kernel(indices, table):
    indices = indices.astype(jnp.int32)
    table = table.astype(jnp.float32)
    counts = _sc_hist(indices.T)
    tablep = jnp.concatenate(
        [table, jnp.zeros((VP - VOCAB, D), jnp.float32)], axis=0)
    return _tc_matmul(counts, tablep)
